# baseline (speedup 1.0000x reference)
import functools

import jax
import jax.numpy as jnp
from jax import lax
from jax.experimental import pallas as pl
from jax.experimental.pallas import tpu as pltpu
from jax.experimental.pallas import tpu_sc as plsc

N = 10000
E = 320000
D = 128

NC = 2
NS = 16
NW = NC * NS
C = 128
NCHUNK = 80
EPW = NCHUNK * C
E_PAD = NW * EPW
NPAD = N + 128
IB = 16
NGRP = NCHUNK // IB
SLAB = 624
TAIL = N - SLAB * (NS - 1)


def _slab_copy(sid, mk_src, mk_dst):
    @pl.when(sid != NS - 1)
    def _():
        base = pl.multiple_of(sid * SLAB, 8)
        pltpu.sync_copy(mk_src(base, SLAB), mk_dst(base, SLAB))

    @pl.when(sid == NS - 1)
    def _():
        base = SLAB * (NS - 1)
        pltpu.sync_copy(mk_src(base, TAIL), mk_dst(base, TAIL))


def _sc_body(h_hbm, src_hbm, dst_hbm, acc_out, src_v, dst_v, rows0_v,
             rows1_v, sem0, sem1, ssem0, ssem1, acc_sp):
    cid = lax.axis_index("c")
    sid = lax.axis_index("s")
    wid = cid * NS + sid
    rows = (rows0_v, rows1_v)
    sems = (sem0, sem1)
    ssems = (ssem0, ssem1)

    _slab_copy(sid, lambda o, s: h_hbm.at[pl.ds(o, s)],
               lambda o, s: acc_sp.at[pl.ds(o, s)])
    plsc.subcore_barrier()

    idx_base = pl.multiple_of(wid * NCHUNK, 8)

    def group(g, carry):
        gb = pl.multiple_of(idx_base + g * IB, 8)
        pltpu.sync_copy(src_hbm.at[pl.ds(gb, IB)], src_v)
        pltpu.sync_copy(dst_hbm.at[pl.ds(gb, IB)], dst_v)
        pend_g = pltpu.async_copy(h_hbm.at[src_v.at[0]], rows[0], sems[0])
        pend_s = [None, None]
        for j in range(IB):
            b = j % 2
            if pend_s[1 - b] is not None:
                pend_s[1 - b].wait()
            if j + 1 < IB:
                nxt = pltpu.async_copy(h_hbm.at[src_v.at[j + 1]],
                                       rows[1 - b], sems[1 - b])
            pend_g.wait()
            pend_s[b] = pltpu.async_copy(rows[b], acc_sp.at[dst_v.at[j]],
                                         ssems[b], add=True)
            if j + 1 < IB:
                pend_g = nxt
        pend_s[(IB - 1) % 2].wait()
        return carry

    lax.fori_loop(0, NGRP, group, 0)
    plsc.subcore_barrier()

    _slab_copy(sid, lambda o, s: acc_sp.at[pl.ds(o, s)],
               lambda o, s: acc_out.at[cid, pl.ds(o, s)])


def _deg_body(dst_hbm, cnt_out, dst_v, ones_v, zb_v, deg_sp):
    cid = lax.axis_index("c")
    sid = lax.axis_index("s")
    wid = cid * NS + sid

    one = jnp.ones((16,), jnp.float32)
    zero = jnp.zeros((16,), jnp.float32)

    def fill(j, carry):
        for k in range(8):
            ones_v[j, pl.ds(k * 16, 16)] = one
            zb_v[j, pl.ds(k * 16, 16)] = zero
        return carry

    lax.fori_loop(0, C, fill, 0)

    @pl.when(sid != NS - 1)
    def _():
        base = pl.multiple_of(sid * SLAB, 8)
        for p in range(4):
            pltpu.sync_copy(zb_v, deg_sp.at[pl.ds(base + p * C, C)])
        pltpu.sync_copy(zb_v.at[pl.ds(0, SLAB - 4 * C)],
                        deg_sp.at[pl.ds(base + 4 * C, SLAB - 4 * C)])

    @pl.when(sid == NS - 1)
    def _():
        base = SLAB * (NS - 1)
        for p in range(5):
            pltpu.sync_copy(zb_v, deg_sp.at[pl.ds(base + p * C, C)])
    plsc.subcore_barrier()

    idx_base = pl.multiple_of(wid * NCHUNK, 8)

    def group(g, carry):
        gb = pl.multiple_of(idx_base + g * IB, 8)
        pltpu.sync_copy(dst_hbm.at[pl.ds(gb, IB)], dst_v)
        for j in range(IB):
            pltpu.sync_copy(ones_v, deg_sp.at[dst_v.at[j]], add=True)
        return carry

    lax.fori_loop(0, NGRP, group, 0)
    plsc.subcore_barrier()

    _slab_copy(sid, lambda o, s: deg_sp.at[pl.ds(o, s)],
               lambda o, s: cnt_out.at[cid, pl.ds(o, s)])


@functools.lru_cache(maxsize=None)
def _build_sc():
    mesh = plsc.VectorSubcoreMesh(core_axis_name="c", subcore_axis_name="s",
                                  num_cores=NC, num_subcores=NS)
    return pl.kernel(
        _sc_body,
        out_type=jax.ShapeDtypeStruct((NC, N, D), jnp.float32),
        mesh=mesh,
        scratch_types=[
            pltpu.VMEM((IB, C), jnp.int32),
            pltpu.VMEM((IB, C), jnp.int32),
            pltpu.VMEM((C, D), jnp.float32),
            pltpu.VMEM((C, D), jnp.float32),
            pltpu.SemaphoreType.DMA,
            pltpu.SemaphoreType.DMA,
            pltpu.SemaphoreType.DMA,
            pltpu.SemaphoreType.DMA,
            pltpu.VMEM_SHARED((NPAD, D), jnp.float32),
        ])


@functools.lru_cache(maxsize=None)
def _build_deg():
    mesh = plsc.VectorSubcoreMesh(core_axis_name="c", subcore_axis_name="s",
                                  num_cores=NC, num_subcores=NS)
    return pl.kernel(
        _deg_body,
        out_type=jax.ShapeDtypeStruct((NC, N, D), jnp.float32),
        mesh=mesh,
        scratch_types=[
            pltpu.VMEM((IB, C), jnp.int32),
            pltpu.VMEM((C, D), jnp.float32),
            pltpu.VMEM((C, D), jnp.float32),
            pltpu.VMEM_SHARED((NPAD, D), jnp.float32),
        ])


def _tc_body(h_ref, a_ref, c_ref, ws_ref, wn_ref,
             b_ref, g_ref, bb_ref, o_ref):
    h = h_ref[...]
    s = a_ref[0] + a_ref[1] - h
    deg = c_ref[0] + c_ref[1] + 1.0
    mean = s / deg
    z = (jnp.dot(h, ws_ref[...], preferred_element_type=jnp.float32)
         + jnp.dot(mean, wn_ref[...], preferred_element_type=jnp.float32)
         + b_ref[...])
    mu = jnp.mean(z, axis=1, keepdims=True)
    d = z - mu
    var = jnp.mean(d * d, axis=1, keepdims=True)
    y = d * lax.rsqrt(var + 1e-5) * g_ref[...] + bb_ref[...]
    o_ref[...] = jnp.where(y > 0, y, jnp.exp(y) - 1.0)


BLK = 2000


def _tc_layer(h, acc, cnt, ws, wn, b, g, bb):
    grid = (N // BLK,)
    row_spec = pl.BlockSpec((BLK, D), lambda i: (i, 0))
    pair_spec = pl.BlockSpec((NC, BLK, D), lambda i: (0, i, 0))
    w_spec = pl.BlockSpec((D, D), lambda i: (0, 0))
    v_spec = pl.BlockSpec((1, D), lambda i: (0, 0))
    return pl.pallas_call(
        _tc_body,
        grid=grid,
        in_specs=[row_spec, pair_spec, pair_spec,
                  w_spec, w_spec, v_spec, v_spec, v_spec],
        out_specs=row_spec,
        out_shape=jax.ShapeDtypeStruct((N, D), jnp.float32),
    )(h, acc, cnt, ws, wn, b.reshape(1, D), g.reshape(1, D),
      bb.reshape(1, D))


def kernel(features, edge_index, W_self1, W_neigh1, b1, ln_g1, ln_b1,
           W_self2, W_neigh2, b2, ln_g2, ln_b2):
    pad = E_PAD - E
    ar = jnp.arange(pad, dtype=edge_index.dtype)
    src = jnp.concatenate([edge_index[0], ar % N]).reshape(NW * NCHUNK, C)
    dst = jnp.concatenate(
        [edge_index[1], N + (ar % (NPAD - N))]
    ).reshape(NW * NCHUNK, C)
    cnt = _build_deg()(dst)
    acc1 = _build_sc()(features, src, dst)
    h1 = _tc_layer(features, acc1, cnt, W_self1, W_neigh1, b1, ln_g1, ln_b1)
    acc2 = _build_sc()(h1, src, dst)
    h2 = _tc_layer(h1, acc2, cnt, W_self2, W_neigh2, b2, ln_g2, ln_b2)
    return h2


# speedup vs baseline: 1.0016x; 1.0016x over previous
import functools

import jax
import jax.numpy as jnp
import numpy as np
from jax import lax
from jax.experimental import pallas as pl
from jax.experimental.pallas import tpu as pltpu
from jax.experimental.pallas import tpu_sc as plsc

N = 10000
E = 320000
D = 128

NC = 2
NS = 16
NW = NC * NS
C = 128
NCHUNK = 80
EPW = NCHUNK * C
E_PAD = NW * EPW
NPAD = N + 128
IB = 16
NGRP = NCHUNK // IB
SLAB = 624
TAIL = N - SLAB * (NS - 1)


def _slab_copy(sid, mk_src, mk_dst):
    @pl.when(sid != NS - 1)
    def _():
        base = pl.multiple_of(sid * SLAB, 8)
        pltpu.sync_copy(mk_src(base, SLAB), mk_dst(base, SLAB))

    @pl.when(sid == NS - 1)
    def _():
        base = SLAB * (NS - 1)
        pltpu.sync_copy(mk_src(base, TAIL), mk_dst(base, TAIL))


def _sc_body(h_hbm, src_hbm, dst_hbm, acc_out, src_v, dst_v, rows0_v,
             rows1_v, sem0, sem1, ssem0, ssem1, acc_sp):
    cid = lax.axis_index("c")
    sid = lax.axis_index("s")
    wid = cid * NS + sid
    rows = (rows0_v, rows1_v)
    sems = (sem0, sem1)
    ssems = (ssem0, ssem1)

    _slab_copy(sid, lambda o, s: h_hbm.at[pl.ds(o, s)],
               lambda o, s: acc_sp.at[pl.ds(o, s)])
    plsc.subcore_barrier()

    idx_base = pl.multiple_of(wid * NCHUNK, 8)

    def group(g, carry):
        gb = pl.multiple_of(idx_base + g * IB, 8)
        pltpu.sync_copy(src_hbm.at[pl.ds(gb, IB)], src_v)
        pltpu.sync_copy(dst_hbm.at[pl.ds(gb, IB)], dst_v)
        pend_g = pltpu.async_copy(h_hbm.at[src_v.at[0]], rows[0], sems[0])
        pend_s = [None, None]
        for j in range(IB):
            b = j % 2
            if pend_s[1 - b] is not None:
                pend_s[1 - b].wait()
            if j + 1 < IB:
                nxt = pltpu.async_copy(h_hbm.at[src_v.at[j + 1]],
                                       rows[1 - b], sems[1 - b])
            pend_g.wait()
            pend_s[b] = pltpu.async_copy(rows[b], acc_sp.at[dst_v.at[j]],
                                         ssems[b], add=True)
            if j + 1 < IB:
                pend_g = nxt
        pend_s[(IB - 1) % 2].wait()
        return carry

    lax.fori_loop(0, NGRP, group, 0)
    plsc.subcore_barrier()

    _slab_copy(sid, lambda o, s: acc_sp.at[pl.ds(o, s)],
               lambda o, s: acc_out.at[cid, pl.ds(o, s)])


def _deg_body(dst_hbm, cnt_out, dst_v, ones_v, zb_v, deg_sp):
    cid = lax.axis_index("c")
    sid = lax.axis_index("s")
    wid = cid * NS + sid

    one = jnp.ones((16,), jnp.float32)
    zero = jnp.zeros((16,), jnp.float32)

    def fill(j, carry):
        for k in range(8):
            ones_v[j, pl.ds(k * 16, 16)] = one
            zb_v[j, pl.ds(k * 16, 16)] = zero
        return carry

    lax.fori_loop(0, C, fill, 0)

    @pl.when(sid != NS - 1)
    def _():
        base = pl.multiple_of(sid * SLAB, 8)
        for p in range(4):
            pltpu.sync_copy(zb_v, deg_sp.at[pl.ds(base + p * C, C)])
        pltpu.sync_copy(zb_v.at[pl.ds(0, SLAB - 4 * C)],
                        deg_sp.at[pl.ds(base + 4 * C, SLAB - 4 * C)])

    @pl.when(sid == NS - 1)
    def _():
        base = SLAB * (NS - 1)
        for p in range(5):
            pltpu.sync_copy(zb_v, deg_sp.at[pl.ds(base + p * C, C)])
    plsc.subcore_barrier()

    idx_base = pl.multiple_of(wid * NCHUNK, 8)

    def group(g, carry):
        gb = pl.multiple_of(idx_base + g * IB, 8)
        pltpu.sync_copy(dst_hbm.at[pl.ds(gb, IB)], dst_v)
        for j in range(IB):
            pltpu.sync_copy(ones_v, deg_sp.at[dst_v.at[j]], add=True)
        return carry

    lax.fori_loop(0, NGRP, group, 0)
    plsc.subcore_barrier()

    _slab_copy(sid, lambda o, s: deg_sp.at[pl.ds(o, s)],
               lambda o, s: cnt_out.at[cid, pl.ds(o, s)])


@functools.lru_cache(maxsize=None)
def _build_sc():
    mesh = plsc.VectorSubcoreMesh(core_axis_name="c", subcore_axis_name="s",
                                  num_cores=NC, num_subcores=NS)
    return pl.kernel(
        _sc_body,
        out_type=jax.ShapeDtypeStruct((NC, N, D), jnp.float32),
        mesh=mesh,
        scratch_types=[
            pltpu.VMEM((IB, C), jnp.int32),
            pltpu.VMEM((IB, C), jnp.int32),
            pltpu.VMEM((C, D), jnp.float32),
            pltpu.VMEM((C, D), jnp.float32),
            pltpu.SemaphoreType.DMA,
            pltpu.SemaphoreType.DMA,
            pltpu.SemaphoreType.DMA,
            pltpu.SemaphoreType.DMA,
            pltpu.VMEM_SHARED((NPAD, D), jnp.float32),
        ])


@functools.lru_cache(maxsize=None)
def _build_deg():
    mesh = plsc.VectorSubcoreMesh(core_axis_name="c", subcore_axis_name="s",
                                  num_cores=NC, num_subcores=NS)
    return pl.kernel(
        _deg_body,
        out_type=jax.ShapeDtypeStruct((NC, N, D), jnp.float32),
        mesh=mesh,
        scratch_types=[
            pltpu.VMEM((IB, C), jnp.int32),
            pltpu.VMEM((C, D), jnp.float32),
            pltpu.VMEM((C, D), jnp.float32),
            pltpu.VMEM_SHARED((NPAD, D), jnp.float32),
        ])


def _tc_body(h_ref, a_ref, c_ref, ws_ref, wn_ref,
             b_ref, g_ref, bb_ref, o_ref):
    h = h_ref[...]
    s = a_ref[0] + a_ref[1] - h
    deg = c_ref[0] + c_ref[1] + 1.0
    mean = s / deg
    z = (jnp.dot(h, ws_ref[...], preferred_element_type=jnp.float32)
         + jnp.dot(mean, wn_ref[...], preferred_element_type=jnp.float32)
         + b_ref[...])
    mu = jnp.mean(z, axis=1, keepdims=True)
    d = z - mu
    var = jnp.mean(d * d, axis=1, keepdims=True)
    y = d * lax.rsqrt(var + 1e-5) * g_ref[...] + bb_ref[...]
    o_ref[...] = jnp.where(y > 0, y, jnp.exp(y) - 1.0)


BLK = 2000


def _tc_layer(h, acc, cnt, ws, wn, b, g, bb):
    grid = (N // BLK,)
    row_spec = pl.BlockSpec((BLK, D), lambda i: (i, 0))
    pair_spec = pl.BlockSpec((NC, BLK, D), lambda i: (0, i, 0))
    w_spec = pl.BlockSpec((D, D), lambda i: (0, 0))
    v_spec = pl.BlockSpec((1, D), lambda i: (0, 0))
    return pl.pallas_call(
        _tc_body,
        grid=grid,
        in_specs=[row_spec, pair_spec, pair_spec,
                  w_spec, w_spec, v_spec, v_spec, v_spec],
        out_specs=row_spec,
        out_shape=jax.ShapeDtypeStruct((N, D), jnp.float32),
    )(h, acc, cnt, ws, wn, b.reshape(1, D), g.reshape(1, D),
      bb.reshape(1, D))


def kernel(features, edge_index, W_self1, W_neigh1, b1, ln_g1, ln_b1,
           W_self2, W_neigh2, b2, ln_g2, ln_b2):
    pad = E_PAD - E
    ar = np.arange(pad, dtype=np.int32)
    pad_src = jnp.asarray(ar % N)
    pad_dst = jnp.asarray(N + (ar % (NPAD - N)))
    src = jnp.concatenate([edge_index[0], pad_src]).reshape(NW * NCHUNK, C)
    dst = jnp.concatenate([edge_index[1], pad_dst]).reshape(NW * NCHUNK, C)
    cnt = _build_deg()(dst)
    acc1 = _build_sc()(features, src, dst)
    h1 = _tc_layer(features, acc1, cnt, W_self1, W_neigh1, b1, ln_g1, ln_b1)
    acc2 = _build_sc()(h1, src, dst)
    h2 = _tc_layer(h1, acc2, cnt, W_self2, W_neigh2, b2, ln_g2, ln_b2)
    return h2


# speedup vs baseline: 1.0256x; 1.0239x over previous
import functools

import jax
import jax.numpy as jnp
import numpy as np
from jax import lax
from jax.experimental import pallas as pl
from jax.experimental.pallas import tpu as pltpu
from jax.experimental.pallas import tpu_sc as plsc

N = 10000
E = 320000
D = 128

NC = 2
NS = 16
NW = NC * NS
C = 128
NCHUNK = 80
EPW = NCHUNK * C
E_PAD = NW * EPW
NPAD = N + 128
IB = 16
NGRP = NCHUNK // IB
SLAB = 624
TAIL = N - SLAB * (NS - 1)


def _slab_copy(sid, mk_src, mk_dst):
    @pl.when(sid != NS - 1)
    def _():
        base = pl.multiple_of(sid * SLAB, 8)
        pltpu.sync_copy(mk_src(base, SLAB), mk_dst(base, SLAB))

    @pl.when(sid == NS - 1)
    def _():
        base = SLAB * (NS - 1)
        pltpu.sync_copy(mk_src(base, TAIL), mk_dst(base, TAIL))


def _sc_body(h_hbm, edges_hbm, acc_out, src_v, dst_v, rows0_v,
             rows1_v, sem0, sem1, ssem0, ssem1, acc_sp):
    cid = lax.axis_index("c")
    sid = lax.axis_index("s")
    wid = cid * NS + sid
    rows = (rows0_v, rows1_v)
    sems = (sem0, sem1)
    ssems = (ssem0, ssem1)

    _slab_copy(sid, lambda o, s: h_hbm.at[pl.ds(o, s)],
               lambda o, s: acc_sp.at[pl.ds(o, s)])
    plsc.subcore_barrier()

    idx_base = pl.multiple_of(wid * NCHUNK, 8)

    def group(g, carry):
        gb = pl.multiple_of(idx_base + g * IB, 8)
        pltpu.sync_copy(edges_hbm.at[0, pl.ds(gb, IB)], src_v)
        pltpu.sync_copy(edges_hbm.at[1, pl.ds(gb, IB)], dst_v)
        pend_g = pltpu.async_copy(h_hbm.at[src_v.at[0]], rows[0], sems[0])
        pend_s = [None, None]
        for j in range(IB):
            b = j % 2
            if pend_s[1 - b] is not None:
                pend_s[1 - b].wait()
            if j + 1 < IB:
                nxt = pltpu.async_copy(h_hbm.at[src_v.at[j + 1]],
                                       rows[1 - b], sems[1 - b])
            pend_g.wait()
            pend_s[b] = pltpu.async_copy(rows[b], acc_sp.at[dst_v.at[j]],
                                         ssems[b], add=True)
            if j + 1 < IB:
                pend_g = nxt
        pend_s[(IB - 1) % 2].wait()
        return carry

    lax.fori_loop(0, NGRP, group, 0)
    plsc.subcore_barrier()

    _slab_copy(sid, lambda o, s: acc_sp.at[pl.ds(o, s)],
               lambda o, s: acc_out.at[cid, pl.ds(o, s)])


def _deg_body(edges_hbm, cnt_out, dst_v, ones_v, zb_v, deg_sp):
    cid = lax.axis_index("c")
    sid = lax.axis_index("s")
    wid = cid * NS + sid

    one = jnp.ones((16,), jnp.float32)
    zero = jnp.zeros((16,), jnp.float32)

    def fill(j, carry):
        for k in range(8):
            ones_v[j, pl.ds(k * 16, 16)] = one
            zb_v[j, pl.ds(k * 16, 16)] = zero
        return carry

    lax.fori_loop(0, C, fill, 0)

    @pl.when(sid != NS - 1)
    def _():
        base = pl.multiple_of(sid * SLAB, 8)
        for p in range(4):
            pltpu.sync_copy(zb_v, deg_sp.at[pl.ds(base + p * C, C)])
        pltpu.sync_copy(zb_v.at[pl.ds(0, SLAB - 4 * C)],
                        deg_sp.at[pl.ds(base + 4 * C, SLAB - 4 * C)])

    @pl.when(sid == NS - 1)
    def _():
        base = SLAB * (NS - 1)
        for p in range(5):
            pltpu.sync_copy(zb_v, deg_sp.at[pl.ds(base + p * C, C)])
    plsc.subcore_barrier()

    idx_base = pl.multiple_of(wid * NCHUNK, 8)

    def group(g, carry):
        gb = pl.multiple_of(idx_base + g * IB, 8)
        pltpu.sync_copy(edges_hbm.at[1, pl.ds(gb, IB)], dst_v)
        for j in range(IB):
            pltpu.sync_copy(ones_v, deg_sp.at[dst_v.at[j]], add=True)
        return carry

    lax.fori_loop(0, NGRP, group, 0)
    plsc.subcore_barrier()

    _slab_copy(sid, lambda o, s: deg_sp.at[pl.ds(o, s)],
               lambda o, s: cnt_out.at[cid, pl.ds(o, s)])


@functools.lru_cache(maxsize=None)
def _build_sc():
    mesh = plsc.VectorSubcoreMesh(core_axis_name="c", subcore_axis_name="s",
                                  num_cores=NC, num_subcores=NS)
    return pl.kernel(
        _sc_body,
        out_type=jax.ShapeDtypeStruct((NC, N, D), jnp.float32),
        mesh=mesh,
        scratch_types=[
            pltpu.VMEM((IB, C), jnp.int32),
            pltpu.VMEM((IB, C), jnp.int32),
            pltpu.VMEM((C, D), jnp.float32),
            pltpu.VMEM((C, D), jnp.float32),
            pltpu.SemaphoreType.DMA,
            pltpu.SemaphoreType.DMA,
            pltpu.SemaphoreType.DMA,
            pltpu.SemaphoreType.DMA,
            pltpu.VMEM_SHARED((NPAD, D), jnp.float32),
        ])


@functools.lru_cache(maxsize=None)
def _build_deg():
    mesh = plsc.VectorSubcoreMesh(core_axis_name="c", subcore_axis_name="s",
                                  num_cores=NC, num_subcores=NS)
    return pl.kernel(
        _deg_body,
        out_type=jax.ShapeDtypeStruct((NC, N, D), jnp.float32),
        mesh=mesh,
        scratch_types=[
            pltpu.VMEM((IB, C), jnp.int32),
            pltpu.VMEM((C, D), jnp.float32),
            pltpu.VMEM((C, D), jnp.float32),
            pltpu.VMEM_SHARED((NPAD, D), jnp.float32),
        ])


def _tc_body(h_ref, a_ref, c_ref, ws_ref, wn_ref,
             b_ref, g_ref, bb_ref, o_ref):
    h = h_ref[...]
    s = a_ref[0] + a_ref[1] - h
    deg = c_ref[0] + c_ref[1] + 1.0
    mean = s / deg
    z = (jnp.dot(h, ws_ref[...], preferred_element_type=jnp.float32)
         + jnp.dot(mean, wn_ref[...], preferred_element_type=jnp.float32)
         + b_ref[...])
    mu = jnp.mean(z, axis=1, keepdims=True)
    d = z - mu
    var = jnp.mean(d * d, axis=1, keepdims=True)
    y = d * lax.rsqrt(var + 1e-5) * g_ref[...] + bb_ref[...]
    o_ref[...] = jnp.where(y > 0, y, jnp.exp(y) - 1.0)


BLK = 2000


def _tc_layer(h, acc, cnt, ws, wn, b, g, bb):
    grid = (N // BLK,)
    row_spec = pl.BlockSpec((BLK, D), lambda i: (i, 0))
    pair_spec = pl.BlockSpec((NC, BLK, D), lambda i: (0, i, 0))
    w_spec = pl.BlockSpec((D, D), lambda i: (0, 0))
    v_spec = pl.BlockSpec((1, D), lambda i: (0, 0))
    return pl.pallas_call(
        _tc_body,
        grid=grid,
        in_specs=[row_spec, pair_spec, pair_spec,
                  w_spec, w_spec, v_spec, v_spec, v_spec],
        out_specs=row_spec,
        out_shape=jax.ShapeDtypeStruct((N, D), jnp.float32),
    )(h, acc, cnt, ws, wn, b.reshape(1, D), g.reshape(1, D),
      bb.reshape(1, D))


def kernel(features, edge_index, W_self1, W_neigh1, b1, ln_g1, ln_b1,
           W_self2, W_neigh2, b2, ln_g2, ln_b2):
    pad = E_PAD - E
    ar = np.arange(pad, dtype=np.int32)
    pads = jnp.asarray(np.stack([ar % N, N + (ar % (NPAD - N))]))
    edges = jnp.concatenate([edge_index, pads], axis=1).reshape(
        2, NW * NCHUNK, C)
    cnt = _build_deg()(edges)
    acc1 = _build_sc()(features, edges)
    h1 = _tc_layer(features, acc1, cnt, W_self1, W_neigh1, b1, ln_g1, ln_b1)
    acc2 = _build_sc()(h1, edges)
    h2 = _tc_layer(h1, acc2, cnt, W_self2, W_neigh2, b2, ln_g2, ln_b2)
    return h2
